# baseline (speedup 1.0000x reference)
import functools

import jax
import jax.numpy as jnp
from jax import lax
from jax.experimental import pallas as pl
from jax.experimental.pallas import tpu as pltpu
from jax.experimental.pallas import tpu_sc as plsc

_NCLS = 10
_B = 1024
_NW = 32
_BPW = _B // _NW
_RING = 8


def _make_kernel():
    mesh = plsc.VectorSubcoreMesh(core_axis_name="c", subcore_axis_name="s")

    @functools.partial(
        pl.kernel,
        mesh=mesh,
        out_type=jax.ShapeDtypeStruct((_B, 100, 784), jnp.float32),
        scratch_types=[
            pltpu.VMEM((_BPW,), jnp.int32),
            pltpu.VMEM_SHARED((_NCLS, 100, 784), jnp.float32),
            pltpu.SemaphoreType.DMA((_RING,)),
        ],
    )
    def emb(y_hbm, a_hbm, out_hbm, y_v, table_s, sems):
        sid = lax.axis_index("s")
        wid = sid * 2 + lax.axis_index("c")
        base = wid * _BPW

        pltpu.sync_copy(y_hbm.at[pl.ds(base, _BPW)], y_v)

        @pl.when(sid < _NCLS)
        def _():
            pltpu.sync_copy(a_hbm.at[pl.ds(sid, 1)], table_s.at[pl.ds(sid, 1)])

        plsc.subcore_barrier()

        def wrblock(row, i, slot):
            return pltpu.make_async_copy(table_s.at[pl.ds(row, 1)],
                                         out_hbm.at[pl.ds(i, 1)],
                                         sems.at[slot])

        def block(g, _):
            y16 = y_v[pl.ds(g * 16, 16)]
            for j in range(16):
                t = g * 16 + j
                row = y16[j]
                slot = lax.rem(t, _RING)

                @pl.when(t >= _RING)
                def _():
                    wrblock(row, base + t, slot).wait()

                wrblock(row, base + t, slot).start()
            return ()

        lax.fori_loop(0, _BPW // 16, block, ())

        for s in range(_RING):
            wrblock(0, 0, s).wait()

    return emb


_emb = _make_kernel()


def kernel(y, A):
    a3 = A.reshape(_NCLS, 100, 784)
    return _emb(y.astype(jnp.int32), a3)
